# Initial kernel scaffold; baseline (speedup 1.0000x reference)
import jax
import jax.numpy as jnp
from jax.experimental import pallas as pl


def kernel(inputs, w0, w, v):
    raise NotImplementedError("write your pallas kernel here")


# speedup vs baseline: 6.0849x; 6.0849x over previous
import functools

import jax
import jax.numpy as jnp
from jax import lax
from jax.experimental import pallas as pl
from jax.experimental.pallas import tpu as pltpu
from jax.experimental.pallas import tpu_sc as plsc

_FIELD = 26
_FEAT = 10000
_K = 8
_D = _FIELD * _K
_BATCH = 4096

_NC = 2
_NS = 16
_NW = _NC * _NS
_RPW = _BATCH // _NW
_CB = 8
_NCHUNK = _RPW // _CB
_IDXN = _CB * _FIELD
_NV = _D // 16


def _dg(x, idx):
    return x.at[idx].get(mode="promise_in_bounds")


def _build():
    mesh = plsc.VectorSubcoreMesh(core_axis_name="c", subcore_axis_name="s")

    @functools.partial(
        pl.kernel,
        mesh=mesh,
        compiler_params=pltpu.CompilerParams(use_tc_tiling_on_sc=False),
        out_type=jax.ShapeDtypeStruct((_BATCH,), jnp.float32),
        scratch_types=[
            pltpu.VMEM((_IDXN,), jnp.int32),
            pltpu.VMEM((_IDXN,), jnp.int32),
            pltpu.VMEM((_IDXN, _D), jnp.float32),
            pltpu.VMEM((_IDXN,), jnp.float32),
            pltpu.VMEM((_RPW,), jnp.float32),
            pltpu.SemaphoreType.DMA,
        ],
    )
    def ffm(in_hbm, w_hbm, v_hbm, out_hbm,
            idx_v, off_v, vrows_v, wv_v, res_v, sem):
        cid = lax.axis_index("c")
        sid = lax.axis_index("s")
        wid = sid * _NC + cid
        base = wid * _RPW
        lane = lax.iota(jnp.int32, 16)

        for j in range(_IDXN // 16):
            i16 = lane + (16 * j)
            off_v[pl.ds(16 * j, 16)] = (i16 % _FIELD) * _FEAT

        for c in range(_NCHUNK):
            row0 = base + c * _CB
            pltpu.sync_copy(in_hbm.at[pl.ds(row0 * _FIELD, _IDXN)], idx_v)
            for j in range(_IDXN // 16):
                sl = pl.ds(16 * j, 16)
                idx_v[sl] = idx_v[sl] + off_v[sl]
            cp_v = pltpu.async_copy(v_hbm.at[idx_v], vrows_v, sem)
            cp_w = pltpu.async_copy(w_hbm.at[idx_v], wv_v, sem)
            cp_v.wait()
            cp_w.wait()

            def row_body(rb, chunkres):
                def f_body(f, accs):
                    r = rb * _FIELD + f
                    return tuple(accs[j] + vrows_v[r, pl.ds(16 * j, 16)]
                                 for j in range(_NV))

                init = tuple(jnp.zeros((16,), jnp.float32)
                             for _ in range(_NV))
                accs = lax.fori_loop(0, _FIELD, f_body, init)

                sq = accs[0] * accs[0]
                s16 = accs[0]
                for j in range(1, _NV):
                    sq = sq + accs[j] * accs[j]
                    s16 = s16 + accs[j]
                t = s16 + _dg(s16, lane ^ 8)

                e0 = rb * _FIELD
                a = pl.multiple_of((e0 // 8) * 8, 8)
                shift = e0 - a
                va = wv_v[pl.ds(a, 16)]
                vb = wv_v[pl.ds(a + 16, 16)]
                i1 = (shift + lane) & 15
                g1 = jnp.where(shift + lane < 16, _dg(va, i1), _dg(vb, i1))
                g2 = jnp.where(lane < 10, _dg(vb, i1),
                               jnp.zeros((16,), jnp.float32))

                combo = g1 + g2 + 0.25 * t * t - 0.5 * sq
                for sh in (8, 4, 2, 1):
                    combo = combo + _dg(combo, lane ^ sh)

                return jnp.where(lane == (rb + 8 * (c % 2)),
                                 combo, chunkres)

            if c % 2 == 0:
                chunkres0 = jnp.zeros((16,), jnp.float32)
            chunkres0 = lax.fori_loop(0, _CB, row_body, chunkres0)
            if c % 2 == 1:
                res_v[pl.ds((c // 2) * 16, 16)] = chunkres0

        pltpu.sync_copy(res_v, out_hbm.at[pl.ds(base, _RPW)])

    return ffm


def kernel(inputs, w0, w, v):
    ffm = _build()
    out = ffm(inputs.reshape(-1), w.reshape(-1), v.reshape(v.shape[0], _D))
    return out.reshape(_BATCH, 1) + w0


# speedup vs baseline: 14.8723x; 2.4441x over previous
import functools

import jax
import jax.numpy as jnp
from jax import lax
from jax.experimental import pallas as pl
from jax.experimental.pallas import tpu as pltpu
from jax.experimental.pallas import tpu_sc as plsc

_FIELD = 26
_FEAT = 10000
_K = 8
_D = _FIELD * _K
_BATCH = 4096

_NC = 2
_NS = 16
_NW = _NC * _NS
_RPW = _BATCH // _NW
_CB = 8
_NCHUNK = _RPW // _CB
_IDXN = _CB * _FIELD
_NV = _D // 16


def _dg(x, idx):
    return x.at[idx].get(mode="promise_in_bounds")


def _build():
    mesh = plsc.VectorSubcoreMesh(core_axis_name="c", subcore_axis_name="s")

    @functools.partial(
        pl.kernel,
        mesh=mesh,
        compiler_params=pltpu.CompilerParams(use_tc_tiling_on_sc=False),
        out_type=jax.ShapeDtypeStruct((_BATCH,), jnp.float32),
        scratch_types=[
            pltpu.VMEM((_IDXN,), jnp.int32),
            pltpu.VMEM((_IDXN,), jnp.int32),
            pltpu.VMEM((_IDXN, _D), jnp.float32),
            pltpu.VMEM((_IDXN,), jnp.float32),
            pltpu.VMEM((_RPW,), jnp.float32),
            pltpu.SemaphoreType.DMA,
        ],
    )
    def ffm(in_hbm, w_hbm, v_hbm, out_hbm,
            idx_v, off_v, vrows_v, wv_v, res_v, sem):
        cid = lax.axis_index("c")
        sid = lax.axis_index("s")
        wid = sid * _NC + cid
        base = wid * _RPW
        lane = lax.iota(jnp.int32, 16)

        for j in range(_IDXN // 16):
            i16 = lane + (16 * j)
            off_v[pl.ds(16 * j, 16)] = (i16 % _FIELD) * _FEAT

        for c in range(_NCHUNK):
            row0 = base + c * _CB
            pltpu.sync_copy(in_hbm.at[pl.ds(row0 * _FIELD, _IDXN)], idx_v)
            for j in range(_IDXN // 16):
                sl = pl.ds(16 * j, 16)
                idx_v[sl] = idx_v[sl] + off_v[sl]
            cp_v = pltpu.async_copy(v_hbm.at[idx_v], vrows_v, sem)
            cp_w = pltpu.async_copy(w_hbm.at[idx_v], wv_v, sem)
            cp_v.wait()
            cp_w.wait()

            def row_body(rb, chunkres):
                def f_body(f, accs):
                    r = rb * _FIELD + f
                    return tuple(accs[j] + vrows_v[r, pl.ds(16 * j, 16)]
                                 for j in range(_NV))

                init = tuple(jnp.zeros((16,), jnp.float32)
                             for _ in range(_NV))
                accs = lax.fori_loop(0, _FIELD, f_body, init)

                sq = accs[0] * accs[0]
                s16 = accs[0]
                for j in range(1, _NV):
                    sq = sq + accs[j] * accs[j]
                    s16 = s16 + accs[j]
                t = s16 + _dg(s16, lane ^ 8)

                e0 = rb * _FIELD
                a = pl.multiple_of((e0 // 8) * 8, 8)
                shift = e0 - a
                va = wv_v[pl.ds(a, 16)]
                vb = wv_v[pl.ds(a + 16, 16)]
                i1 = (shift + lane) & 15
                g1 = jnp.where(shift + lane < 16, _dg(va, i1), _dg(vb, i1))
                g2 = jnp.where(lane < 10, _dg(vb, i1),
                               jnp.zeros((16,), jnp.float32))

                combo = g1 + g2 + 0.25 * t * t - 0.5 * sq
                for sh in (8, 4, 2, 1):
                    combo = combo + _dg(combo, lane ^ sh)

                return jnp.where(lane == (rb + 8 * (c % 2)),
                                 combo, chunkres)

            if c % 2 == 0:
                chunkres0 = jnp.zeros((16,), jnp.float32)
            chunkres0 = lax.fori_loop(0, _CB, row_body, chunkres0)
            if c % 2 == 1:
                res_v[pl.ds((c // 2) * 16, 16)] = chunkres0

        pltpu.sync_copy(res_v, out_hbm.at[pl.ds(base, _RPW)])

    return ffm


_TBLK = 2048


def _tr_body(vt_ref, out_ref):
    out_ref[...] = vt_ref[...].T


def _to_row_major(v):
    rows = v.shape[0]
    vt = jnp.transpose(v, (1, 2, 0)).reshape(_D, rows)
    return pl.pallas_call(
        _tr_body,
        grid=(pl.cdiv(rows, _TBLK),),
        in_specs=[pl.BlockSpec((_D, _TBLK), lambda j: (0, j))],
        out_specs=pl.BlockSpec((_TBLK, _D), lambda j: (j, 0)),
        out_shape=jax.ShapeDtypeStruct((rows, _D), jnp.float32),
    )(vt)


def kernel(inputs, w0, w, v):
    ffm = _build()
    out = ffm(inputs.reshape(-1), w.reshape(-1), _to_row_major(v))
    return out.reshape(_BATCH, 1) + w0


# speedup vs baseline: 29.5914x; 1.9897x over previous
import functools

import jax
import jax.numpy as jnp
from jax import lax
from jax.experimental import pallas as pl
from jax.experimental.pallas import tpu as pltpu
from jax.experimental.pallas import tpu_sc as plsc

_FIELD = 26
_FEAT = 10000
_K = 8
_D = _FIELD * _K
_BATCH = 4096

_NC = 2
_NS = 16
_NW = _NC * _NS
_RPW = _BATCH // _NW
_CB = 8
_NCHUNK = _RPW // _CB
_IDXN = _CB * _FIELD
_NV = _D // 16


def _dg(x, idx):
    return x.at[idx].get(mode="promise_in_bounds")


def _build():
    mesh = plsc.VectorSubcoreMesh(core_axis_name="c", subcore_axis_name="s")

    @functools.partial(
        pl.kernel,
        mesh=mesh,
        compiler_params=pltpu.CompilerParams(use_tc_tiling_on_sc=False),
        out_type=jax.ShapeDtypeStruct((_BATCH,), jnp.float32),
        scratch_types=[
            pltpu.VMEM((_IDXN,), jnp.int32),
            pltpu.VMEM((_IDXN,), jnp.int32),
            pltpu.VMEM((_IDXN, 128), jnp.float32),
            pltpu.VMEM((_IDXN, 128), jnp.float32),
            pltpu.VMEM((_IDXN,), jnp.float32),
            pltpu.VMEM((_RPW,), jnp.float32),
            pltpu.SemaphoreType.DMA,
        ],
    )
    def ffm(in_hbm, w_hbm, va_hbm, vb_hbm, out_hbm,
            idx_v, off_v, vra_v, vrb_v, wv_v, res_v, sem):
        cid = lax.axis_index("c")
        sid = lax.axis_index("s")
        wid = sid * _NC + cid
        base = wid * _RPW
        lane = lax.iota(jnp.int32, 16)

        for j in range(_IDXN // 16):
            i16 = lane + (16 * j)
            off_v[pl.ds(16 * j, 16)] = (i16 % _FIELD) * _FEAT

        for c in range(_NCHUNK):
            row0 = base + c * _CB
            pltpu.sync_copy(in_hbm.at[pl.ds(row0 * _FIELD, _IDXN)], idx_v)
            for j in range(_IDXN // 16):
                sl = pl.ds(16 * j, 16)
                idx_v[sl] = idx_v[sl] + off_v[sl]
            cp_a = pltpu.async_copy(va_hbm.at[idx_v], vra_v, sem)
            cp_b = pltpu.async_copy(vb_hbm.at[idx_v], vrb_v, sem)
            cp_w = pltpu.async_copy(w_hbm.at[idx_v], wv_v, sem)
            cp_a.wait()
            cp_b.wait()
            cp_w.wait()

            def row_body(rb, chunkres):
                def f_body(f, accs):
                    r = rb * _FIELD + f
                    return tuple(
                        (accs[j] + vra_v[r, pl.ds(16 * j, 16)]) if j < 8
                        else (accs[j] + vrb_v[r, pl.ds(16 * (j - 8), 16)])
                        for j in range(_NV))

                init = tuple(jnp.zeros((16,), jnp.float32)
                             for _ in range(_NV))
                accs = lax.fori_loop(0, _FIELD, f_body, init)

                sq = accs[0] * accs[0]
                s16 = accs[0]
                for j in range(1, _NV):
                    sq = sq + accs[j] * accs[j]
                    s16 = s16 + accs[j]
                t = s16 + _dg(s16, lane ^ 8)

                e0 = rb * _FIELD
                a = pl.multiple_of((e0 // 8) * 8, 8)
                shift = e0 - a
                va = wv_v[pl.ds(a, 16)]
                vb = wv_v[pl.ds(a + 16, 16)]
                i1 = (shift + lane) & 15
                g1 = jnp.where(shift + lane < 16, _dg(va, i1), _dg(vb, i1))
                g2 = jnp.where(lane < 10, _dg(vb, i1),
                               jnp.zeros((16,), jnp.float32))

                combo = g1 + g2 + 0.25 * t * t - 0.5 * sq
                for sh in (8, 4, 2, 1):
                    combo = combo + _dg(combo, lane ^ sh)

                return jnp.where(lane == (rb + 8 * (c % 2)),
                                 combo, chunkres)

            if c % 2 == 0:
                chunkres0 = jnp.zeros((16,), jnp.float32)
            chunkres0 = lax.fori_loop(0, _CB, row_body, chunkres0)
            if c % 2 == 1:
                res_v[pl.ds((c // 2) * 16, 16)] = chunkres0

        pltpu.sync_copy(res_v, out_hbm.at[pl.ds(base, _RPW)])

    return ffm


_TBLK = 2048


def _tr_body(vt_ref, a_ref, b_ref):
    xp = jnp.concatenate(
        [vt_ref[...], jnp.zeros((256 - _D, _TBLK), jnp.float32)], axis=0)
    xt = xp.T
    a_ref[...] = xt[:, :128]
    b_ref[...] = xt[:, 128:]


def _to_row_major(v):
    rows = v.shape[0]
    vt = jnp.transpose(v, (1, 2, 0)).reshape(_D, rows)
    return pl.pallas_call(
        _tr_body,
        grid=(pl.cdiv(rows, _TBLK),),
        in_specs=[pl.BlockSpec((_D, _TBLK), lambda j: (0, j))],
        out_specs=[pl.BlockSpec((_TBLK, 128), lambda j: (j, 0)),
                   pl.BlockSpec((_TBLK, 128), lambda j: (j, 0))],
        out_shape=[jax.ShapeDtypeStruct((rows, 128), jnp.float32),
                   jax.ShapeDtypeStruct((rows, 128), jnp.float32)],
    )(vt)


def kernel(inputs, w0, w, v):
    ffm = _build()
    va, vb = _to_row_major(v)
    out = ffm(inputs.reshape(-1), w.reshape(-1), va, vb)
    return out.reshape(_BATCH, 1) + w0


# speedup vs baseline: 37.9544x; 1.2826x over previous
import functools

import jax
import jax.numpy as jnp
from jax import lax
from jax.experimental import pallas as pl
from jax.experimental.pallas import tpu as pltpu
from jax.experimental.pallas import tpu_sc as plsc

_FIELD = 26
_FEAT = 10000
_K = 8
_D = _FIELD * _K
_BATCH = 4096

_NC = 2
_NS = 16
_NW = _NC * _NS
_RPW = _BATCH // _NW
_CB = 8
_NCHUNK = _RPW // _CB
_IDXN = _CB * _FIELD
_NV = _D // 16


def _dg(x, idx):
    return x.at[idx].get(mode="promise_in_bounds")


def _build():
    mesh = plsc.VectorSubcoreMesh(core_axis_name="c", subcore_axis_name="s")

    @functools.partial(
        pl.kernel,
        mesh=mesh,
        compiler_params=pltpu.CompilerParams(use_tc_tiling_on_sc=False),
        out_type=jax.ShapeDtypeStruct((_BATCH,), jnp.float32),
        scratch_types=[
            pltpu.VMEM((_IDXN,), jnp.int32),
            pltpu.VMEM((_IDXN,), jnp.int32),
            pltpu.VMEM((_IDXN,), jnp.int32),
            pltpu.VMEM((_IDXN, 128), jnp.float32),
            pltpu.VMEM((_IDXN, 128), jnp.float32),
            pltpu.VMEM((_IDXN, 128), jnp.float32),
            pltpu.VMEM((_IDXN, 128), jnp.float32),
            pltpu.VMEM((_IDXN,), jnp.float32),
            pltpu.VMEM((_IDXN,), jnp.float32),
            pltpu.VMEM((_RPW,), jnp.float32),
            pltpu.SemaphoreType.DMA,
            pltpu.SemaphoreType.DMA,
        ],
    )
    def ffm(in_hbm, w_hbm, va_hbm, vb_hbm, out_hbm,
            idx0_v, idx1_v, off_v, vra0_v, vra1_v, vrb0_v, vrb1_v,
            wv0_v, wv1_v, res_v, sem0, sem1):
        cid = lax.axis_index("c")
        sid = lax.axis_index("s")
        wid = sid * _NC + cid
        base = wid * _RPW
        lane = lax.iota(jnp.int32, 16)

        for j in range(_IDXN // 16):
            i16 = lane + (16 * j)
            off_v[pl.ds(16 * j, 16)] = (i16 % _FIELD) * _FEAT

        bufs = [(idx0_v, vra0_v, vrb0_v, wv0_v, sem0),
                (idx1_v, vra1_v, vrb1_v, wv1_v, sem1)]

        def stage(c):
            idx_v, vra_v, vrb_v, wv_v, sem = bufs[c % 2]
            row0 = base + c * _CB
            pltpu.sync_copy(in_hbm.at[pl.ds(row0 * _FIELD, _IDXN)], idx_v)
            for j in range(_IDXN // 16):
                sl = pl.ds(16 * j, 16)
                idx_v[sl] = idx_v[sl] + off_v[sl]
            return (pltpu.async_copy(va_hbm.at[idx_v], vra_v, sem),
                    pltpu.async_copy(vb_hbm.at[idx_v], vrb_v, sem),
                    pltpu.async_copy(w_hbm.at[idx_v], wv_v, sem))

        cps = stage(0)
        for c in range(_NCHUNK):
            _, vra_v, vrb_v, wv_v, _ = bufs[c % 2]
            nxt = stage(c + 1) if c + 1 < _NCHUNK else None
            for cp in cps:
                cp.wait()
            cps = nxt

            def row_body(rb, chunkres):
                def f_body(f, accs):
                    r = rb * _FIELD + f
                    return tuple(
                        (accs[j] + vra_v[r, pl.ds(16 * j, 16)]) if j < 8
                        else (accs[j] + vrb_v[r, pl.ds(16 * (j - 8), 16)])
                        for j in range(_NV))

                init = tuple(jnp.zeros((16,), jnp.float32)
                             for _ in range(_NV))
                accs = lax.fori_loop(0, _FIELD, f_body, init)

                sq = accs[0] * accs[0]
                s16 = accs[0]
                for j in range(1, _NV):
                    sq = sq + accs[j] * accs[j]
                    s16 = s16 + accs[j]
                t = s16 + _dg(s16, lane ^ 8)

                e0 = rb * _FIELD
                a = pl.multiple_of((e0 // 8) * 8, 8)
                shift = e0 - a
                va = wv_v[pl.ds(a, 16)]
                vb = wv_v[pl.ds(a + 16, 16)]
                i1 = (shift + lane) & 15
                g1 = jnp.where(shift + lane < 16, _dg(va, i1), _dg(vb, i1))
                g2 = jnp.where(lane < 10, _dg(vb, i1),
                               jnp.zeros((16,), jnp.float32))

                combo = g1 + g2 + 0.25 * t * t - 0.5 * sq
                for sh in (8, 4, 2, 1):
                    combo = combo + _dg(combo, lane ^ sh)

                return jnp.where(lane == (rb + 8 * (c % 2)),
                                 combo, chunkres)

            if c % 2 == 0:
                chunkres0 = jnp.zeros((16,), jnp.float32)
            chunkres0 = lax.fori_loop(0, _CB, row_body, chunkres0)
            if c % 2 == 1:
                res_v[pl.ds((c // 2) * 16, 16)] = chunkres0

        pltpu.sync_copy(res_v, out_hbm.at[pl.ds(base, _RPW)])

    return ffm


_TBLK = 4096


def _tr_body(vt_ref, a_ref, b_ref):
    xp = jnp.concatenate(
        [vt_ref[...], jnp.zeros((256 - _D, _TBLK), jnp.float32)], axis=0)
    xt = xp.T
    a_ref[...] = xt[:, :128]
    b_ref[...] = xt[:, 128:]


def _to_row_major(v):
    rows = v.shape[0]
    vt = jnp.transpose(v, (1, 2, 0)).reshape(_D, rows)
    return pl.pallas_call(
        _tr_body,
        grid=(pl.cdiv(rows, _TBLK),),
        in_specs=[pl.BlockSpec((_D, _TBLK), lambda j: (0, j))],
        out_specs=[pl.BlockSpec((_TBLK, 128), lambda j: (j, 0)),
                   pl.BlockSpec((_TBLK, 128), lambda j: (j, 0))],
        out_shape=[jax.ShapeDtypeStruct((rows, 128), jnp.float32),
                   jax.ShapeDtypeStruct((rows, 128), jnp.float32)],
    )(vt)


def kernel(inputs, w0, w, v):
    ffm = _build()
    va, vb = _to_row_major(v)
    out = ffm(inputs.reshape(-1), w.reshape(-1), va, vb)
    return out.reshape(_BATCH, 1) + w0


# speedup vs baseline: 39.3124x; 1.0358x over previous
import functools

import jax
import jax.numpy as jnp
from jax import lax
from jax.experimental import pallas as pl
from jax.experimental.pallas import tpu as pltpu
from jax.experimental.pallas import tpu_sc as plsc

_FIELD = 26
_FEAT = 10000
_K = 8
_D = _FIELD * _K
_BATCH = 4096

_NC = 2
_NS = 16
_NW = _NC * _NS
_RPW = _BATCH // _NW
_CB = 8
_NCHUNK = _RPW // _CB
_IDXN = _CB * _FIELD
_NV = _D // 16


def _dg(x, idx):
    return x.at[idx].get(mode="promise_in_bounds")


def _build():
    mesh = plsc.VectorSubcoreMesh(core_axis_name="c", subcore_axis_name="s")

    @functools.partial(
        pl.kernel,
        mesh=mesh,
        compiler_params=pltpu.CompilerParams(use_tc_tiling_on_sc=False),
        out_type=jax.ShapeDtypeStruct((_BATCH,), jnp.float32),
        scratch_types=[
            pltpu.VMEM((_IDXN,), jnp.int32),
            pltpu.VMEM((_IDXN,), jnp.int32),
            pltpu.VMEM((_IDXN,), jnp.int32),
            pltpu.VMEM((_IDXN, 128), jnp.float32),
            pltpu.VMEM((_IDXN, 128), jnp.float32),
            pltpu.VMEM((_IDXN, 128), jnp.float32),
            pltpu.VMEM((_IDXN, 128), jnp.float32),
            pltpu.VMEM((_IDXN,), jnp.float32),
            pltpu.VMEM((_IDXN,), jnp.float32),
            pltpu.VMEM((_RPW,), jnp.float32),
            pltpu.SemaphoreType.DMA,
            pltpu.SemaphoreType.DMA,
        ],
    )
    def ffm(in_hbm, w_hbm, va_hbm, vb_hbm, out_hbm,
            idx0_v, idx1_v, off_v, vra0_v, vra1_v, vrb0_v, vrb1_v,
            wv0_v, wv1_v, res_v, sem0, sem1):
        cid = lax.axis_index("c")
        sid = lax.axis_index("s")
        wid = sid * _NC + cid
        base = wid * _RPW
        lane = lax.iota(jnp.int32, 16)

        for j in range(_IDXN // 16):
            i16 = lane + (16 * j)
            off_v[pl.ds(16 * j, 16)] = (i16 % _FIELD) * _FEAT

        bufs = [(idx0_v, vra0_v, vrb0_v, wv0_v, sem0),
                (idx1_v, vra1_v, vrb1_v, wv1_v, sem1)]

        def stage(c):
            idx_v, vra_v, vrb_v, wv_v, sem = bufs[c % 2]
            row0 = base + c * _CB
            pltpu.sync_copy(in_hbm.at[pl.ds(row0 * _FIELD, _IDXN)], idx_v)
            for j in range(_IDXN // 16):
                sl = pl.ds(16 * j, 16)
                idx_v[sl] = idx_v[sl] + off_v[sl]
            return (pltpu.async_copy(va_hbm.at[idx_v], vra_v, sem),
                    pltpu.async_copy(vb_hbm.at[idx_v], vrb_v, sem),
                    pltpu.async_copy(w_hbm.at[idx_v], wv_v, sem))

        cps = stage(0)
        for c in range(_NCHUNK):
            _, vra_v, vrb_v, wv_v, _ = bufs[c % 2]
            nxt = stage(c + 1) if c + 1 < _NCHUNK else None
            for cp in cps:
                cp.wait()
            cps = nxt

            def row_body(rb, chunkres):
                def f_body(f, accs):
                    r = rb * _FIELD + f
                    return tuple(
                        (accs[j] + vra_v[r, pl.ds(16 * j, 16)]) if j < 8
                        else (accs[j] + vrb_v[r, pl.ds(16 * (j - 8), 16)])
                        for j in range(_NV))

                init = tuple(jnp.zeros((16,), jnp.float32)
                             for _ in range(_NV))
                accs = lax.fori_loop(0, _FIELD, f_body, init)

                sq = accs[0] * accs[0]
                s16 = accs[0]
                for j in range(1, _NV):
                    sq = sq + accs[j] * accs[j]
                    s16 = s16 + accs[j]
                t = s16 + _dg(s16, lane ^ 8)

                e0 = rb * _FIELD
                a = pl.multiple_of((e0 // 8) * 8, 8)
                shift = e0 - a
                va = wv_v[pl.ds(a, 16)]
                vb = wv_v[pl.ds(a + 16, 16)]
                i1 = (shift + lane) & 15
                g1 = jnp.where(shift + lane < 16, _dg(va, i1), _dg(vb, i1))
                g2 = jnp.where(lane < 10, _dg(vb, i1),
                               jnp.zeros((16,), jnp.float32))

                combo = g1 + g2 + 0.25 * t * t - 0.5 * sq
                for sh in (8, 4, 2, 1):
                    combo = combo + _dg(combo, lane ^ sh)

                return jnp.where(lane == (rb + 8 * (c % 2)),
                                 combo, chunkres)

            if c % 2 == 0:
                chunkres0 = jnp.zeros((16,), jnp.float32)
            chunkres0 = lax.fori_loop(0, _CB, row_body, chunkres0)
            if c % 2 == 1:
                res_v[pl.ds((c // 2) * 16, 16)] = chunkres0

        pltpu.sync_copy(res_v, out_hbm.at[pl.ds(base, _RPW)])

    return ffm


_TBLK = 8192


def _tr_body(vt_ref, a_ref, b_ref):
    xp = jnp.concatenate(
        [vt_ref[...], jnp.zeros((256 - _D, _TBLK), jnp.float32)], axis=0)
    xt = xp.T
    a_ref[...] = xt[:, :128]
    b_ref[...] = xt[:, 128:]


def _to_row_major(v):
    rows = v.shape[0]
    vt = jnp.transpose(v, (1, 2, 0)).reshape(_D, rows)
    return pl.pallas_call(
        _tr_body,
        grid=(pl.cdiv(rows, _TBLK),),
        in_specs=[pl.BlockSpec((_D, _TBLK), lambda j: (0, j))],
        out_specs=[pl.BlockSpec((_TBLK, 128), lambda j: (j, 0)),
                   pl.BlockSpec((_TBLK, 128), lambda j: (j, 0))],
        out_shape=[jax.ShapeDtypeStruct((rows, 128), jnp.float32),
                   jax.ShapeDtypeStruct((rows, 128), jnp.float32)],
    )(vt)


def kernel(inputs, w0, w, v):
    ffm = _build()
    va, vb = _to_row_major(v)
    out = ffm(inputs.reshape(-1), w.reshape(-1), va, vb)
    return out.reshape(_BATCH, 1) + w0
